# baseline (speedup 1.0000x reference)
import jax
import jax.numpy as jnp
from jax import lax
from jax.experimental import pallas as pl
from jax.experimental.pallas import tpu as pltpu
from jax.experimental.pallas import tpu_sc as plsc

_NC = 2
_NS = 16
_NW = _NC * _NS

_B = 4096
_L = 20
_D = 300
_BPW = _B // _NW
_EG = 4
_NG = _BPW // _EG
_RG = _EG * _L
_CHUNK_OFFS = tuple(j * 16 for j in range(_D // 16)) + (_D - 16,)
_NCH = len(_CHUNK_OFFS)
_INV_L = 1.0 / _L


def _pool_body(x_hbm, tab_hbm, out_hbm, xv, rows0, rows1, outv, sem0, sem1):
    wid = lax.axis_index("s") * _NC + lax.axis_index("c")
    base = wid * _BPW
    pltpu.sync_copy(x_hbm.at[pl.ds(base * _L, _BPW * _L)], xv)

    def idx_slice(g):
        off = pl.multiple_of(g * _RG, 8)
        return xv.at[pl.ds(off, _RG)]

    def accumulate(rows, g):
        for e in range(_EG):
            def rbody(l, accs, e=e):
                r = e * _L + l
                return tuple(accs[j] + rows[r, pl.ds(_CHUNK_OFFS[j], 16)]
                             for j in range(_NCH))
            accs = lax.fori_loop(
                0, _L, rbody,
                tuple(jnp.zeros((16,), jnp.float32) for _ in range(_NCH)))
            row_out = g * _EG + e
            for j in range(_NCH):
                outv[row_out, pl.ds(_CHUNK_OFFS[j], 16)] = accs[j] * _INV_L

    pltpu.async_copy(tab_hbm.at[idx_slice(0)], rows0, sem0)
    pltpu.async_copy(tab_hbm.at[idx_slice(1)], rows1, sem1)

    def gbody(g2, carry):
        g = g2 * 2
        pltpu.make_async_copy(tab_hbm.at[idx_slice(g)], rows0, sem0).wait()
        accumulate(rows0, g)
        pltpu.async_copy(tab_hbm.at[idx_slice(g + 2)], rows0, sem0)
        pltpu.make_async_copy(tab_hbm.at[idx_slice(g + 1)], rows1, sem1).wait()
        accumulate(rows1, g + 1)
        pltpu.async_copy(tab_hbm.at[idx_slice(g + 3)], rows1, sem1)
        return carry

    lax.fori_loop(0, _NG // 2 - 1, gbody, 0)

    g_last = _NG - 2
    pltpu.make_async_copy(tab_hbm.at[idx_slice(g_last)], rows0, sem0).wait()
    accumulate(rows0, g_last)
    pltpu.make_async_copy(tab_hbm.at[idx_slice(g_last + 1)], rows1, sem1).wait()
    accumulate(rows1, g_last + 1)

    pltpu.sync_copy(outv, out_hbm.at[pl.ds(base, _BPW)])


_pool = pl.kernel(
    _pool_body,
    out_type=jax.ShapeDtypeStruct((_B, _D), jnp.float32),
    mesh=plsc.VectorSubcoreMesh(
        core_axis_name="c", subcore_axis_name="s",
        num_cores=_NC, num_subcores=_NS),
    scratch_types=[
        pltpu.VMEM((_BPW * _L,), jnp.int32),
        pltpu.VMEM((_RG, _D), jnp.float32),
        pltpu.VMEM((_RG, _D), jnp.float32),
        pltpu.VMEM((_BPW, _D), jnp.float32),
        pltpu.SemaphoreType.DMA,
        pltpu.SemaphoreType.DMA,
    ],
    compiler_params=pltpu.CompilerParams(use_tc_tiling_on_sc=False),
)


def _mlp_body(h_ref, w1_ref, b1_ref, w2_ref, b2_ref, w3_ref, b3_ref, o_ref):
    h = h_ref[...]
    h1 = jnp.maximum(
        jnp.dot(h, w1_ref[...], preferred_element_type=jnp.float32)
        + b1_ref[...], 0.0)
    h2 = jnp.maximum(
        jnp.dot(h1, w2_ref[...], preferred_element_type=jnp.float32)
        + b2_ref[...], 0.0)
    o_ref[...] = (
        jnp.dot(h2, w3_ref[...], preferred_element_type=jnp.float32)
        + b3_ref[...])


_MLP_BLK = 512


def _mlp(pooled, W1, b1, W2, b2, W3, b3):
    grid = (_B // _MLP_BLK,)
    return pl.pallas_call(
        _mlp_body,
        out_shape=jax.ShapeDtypeStruct((_B, 1), jnp.float32),
        grid=grid,
        in_specs=[
            pl.BlockSpec((_MLP_BLK, _D), lambda i: (i, 0)),
            pl.BlockSpec(W1.shape, lambda i: (0, 0)),
            pl.BlockSpec(b1.shape, lambda i: (0, 0)),
            pl.BlockSpec(W2.shape, lambda i: (0, 0)),
            pl.BlockSpec(b2.shape, lambda i: (0, 0)),
            pl.BlockSpec(W3.shape, lambda i: (0, 0)),
            pl.BlockSpec(b3.shape, lambda i: (0, 0)),
        ],
        out_specs=pl.BlockSpec((_MLP_BLK, 1), lambda i: (i, 0)),
    )(pooled, W1, b1, W2, b2, W3, b3)


def kernel(x, table, W1, b1, W2, b2, W3, b3):
    x_flat = x.reshape(-1).astype(jnp.int32)
    pooled = _pool(x_flat, table)
    return _mlp(pooled, W1, b1.reshape(1, -1), W2, b2.reshape(1, -1),
                W3, b3.reshape(1, -1))
